# baseline (speedup 1.0000x reference)
import functools

import jax
import jax.numpy as jnp
from jax import lax
from jax.experimental import pallas as pl
from jax.experimental.pallas import tpu as pltpu
from jax.experimental.pallas import tpu_sc as plsc

N = 10000
E = 320000
D = 128

NC = 2
NS = 16
NW = NC * NS
CHUNK = 112
NP = 10240
ROWS_PT = NP // NS
NBUF = 3
K = NBUF - 1
NIB = 2 * NBUF
NCHUNK = 90
EPT = NCHUNK * CHUNK
E_PAD = EPT * NW
NROWI = E_PAD // CHUNK

_MESH = dict(core_axis_name="c", subcore_axis_name="s", num_cores=NC,
             num_subcores=NS)


def _conv_impl(with_deg, p_hbm, idx2_hbm, zrows_hbm, zrow_hbm,
               acc_hbm, deg_hbm,
               acc_sp, deg_sp, idx_ring, rows, ones_v, sem_g, sem_s, sem_i):
    c = lax.axis_index("c")
    s = lax.axis_index("s")
    ebase = (c * NS + s) * NCHUNK

    pltpu.sync_copy(zrows_hbm, acc_sp.at[pl.ds(s * ROWS_PT, ROWS_PT)])
    if with_deg:
        pltpu.sync_copy(zrow_hbm, deg_sp.at[pl.ds(s * ROWS_PT, ROWS_PT)])
        for j in range(CHUNK // 16):
            ones_v[pl.ds(j * 16, 16)] = jnp.full((16,), 1.0, jnp.float32)
    plsc.subcore_barrier()

    def prefetch_idx(jj, slot):
        pltpu.async_copy(idx2_hbm.at[ebase + jj], idx_ring.at[slot],
                         sem_i[slot])

    def issue_gather(jj_unused, b, slot):
        pltpu.make_async_copy(idx2_hbm.at[ebase], idx_ring.at[slot],
                              sem_i[slot]).wait()
        pltpu.async_copy(p_hbm.at[idx_ring.at[slot, 0]], rows[b], sem_g[b])

    def wait_scatter(b, slot):
        pltpu.make_async_copy(rows[b], acc_sp.at[idx_ring.at[slot, 1]],
                              sem_s[b]).wait()
        if with_deg:
            pltpu.make_async_copy(ones_v, deg_sp.at[idx_ring.at[slot, 1]],
                                  sem_s[b]).wait()

    def step(jj, u, drain, pref, gath):
        b = u % NBUF
        if drain:
            wait_scatter((u - 1) % NBUF, (u - 1) % NIB)
        if pref:
            prefetch_idx(jj + NBUF, (u + NBUF) % NIB)
        pltpu.make_async_copy(p_hbm.at[idx_ring.at[u, 0]], rows[b],
                              sem_g[b]).wait()
        pltpu.async_copy(rows[b], acc_sp.at[idx_ring.at[u, 1]], sem_s[b],
                         add=True)
        if with_deg:
            pltpu.async_copy(ones_v, deg_sp.at[idx_ring.at[u, 1]], sem_s[b],
                             add=True)
        if gath:
            issue_gather(jj + K, (b + K) % NBUF, (u + K) % NIB)

    for m in range(NBUF):
        prefetch_idx(m, m)
    for b in range(K):
        issue_gather(b, b, b)

    for u in range(NIB):
        step(u, u, drain=(u > 0), pref=True, gath=True)

    def round_body(r, _):
        jj0 = r * NIB
        for u in range(NIB):
            step(jj0 + u, u, drain=True, pref=True, gath=True)
        return 0

    lax.fori_loop(1, NCHUNK // NIB - 1, round_body, 0)

    jl = NCHUNK - NIB
    for u in range(NIB):
        step(jl + u, u, drain=True, pref=(u < NIB - NBUF),
             gath=(u < NIB - K))
    wait_scatter((NIB - 1) % NBUF, NIB - 1)
    plsc.subcore_barrier()

    r = pl.ds(s * ROWS_PT, ROWS_PT)
    pltpu.sync_copy(acc_sp.at[r], acc_hbm.at[c, r])
    if with_deg:
        pltpu.sync_copy(deg_sp.at[r], deg_hbm.at[c, r])


def _common_scratch():
    return [
        pltpu.VMEM((NIB, 2, CHUNK), jnp.int32),
        [pltpu.VMEM((CHUNK, D), jnp.float32) for _ in range(NBUF)],
        [pltpu.SemaphoreType.DMA for _ in range(NBUF)],
        [pltpu.SemaphoreType.DMA for _ in range(NBUF)],
        [pltpu.SemaphoreType.DMA for _ in range(NIB)],
    ]


@functools.partial(
    pl.kernel,
    out_type=(jax.ShapeDtypeStruct((NC, NP, D), jnp.float32),
              jax.ShapeDtypeStruct((NC, NP), jnp.float32)),
    mesh=plsc.VectorSubcoreMesh(**_MESH),
    scratch_types=[pltpu.VMEM_SHARED((NP, D), jnp.float32),
                   pltpu.VMEM_SHARED((NP,), jnp.float32),
                   pltpu.VMEM((CHUNK,), jnp.float32)] + _common_scratch(),
)
def _sc_conv_deg(p_hbm, idx2_hbm, zrows_hbm, zrow_hbm, acc_hbm, deg_hbm,
                 acc_sp, deg_sp, ones_v, idx_ring, rows, sem_g, sem_s, sem_i):
    _conv_impl(True, p_hbm, idx2_hbm, zrows_hbm, zrow_hbm, acc_hbm, deg_hbm,
               acc_sp, deg_sp, idx_ring, rows, ones_v, sem_g, sem_s, sem_i)


@functools.partial(
    pl.kernel,
    out_type=jax.ShapeDtypeStruct((NC, NP, D), jnp.float32),
    mesh=plsc.VectorSubcoreMesh(**_MESH),
    scratch_types=[pltpu.VMEM_SHARED((NP, D), jnp.float32)]
    + _common_scratch(),
)
def _sc_conv(p_hbm, idx2_hbm, zrows_hbm, acc_hbm,
             acc_sp, idx_ring, rows, sem_g, sem_s, sem_i):
    _conv_impl(False, p_hbm, idx2_hbm, zrows_hbm, None, acc_hbm, None,
               acc_sp, None, idx_ring, rows, None, sem_g, sem_s, sem_i)


_R = 1280


def _pre_body(x_ref, wpre_ref, bpre_ref, w1_ref, out_ref):
    h = jnp.dot(x_ref[...], wpre_ref[...], preferred_element_type=jnp.float32)
    h = h + bpre_ref[...]
    out_ref[...] = jnp.dot(h, w1_ref[...], preferred_element_type=jnp.float32)


def _tc_pre(x_pad, w_pre, b_pre, w1):
    return pl.pallas_call(
        _pre_body,
        grid=(NP // _R,),
        in_specs=[
            pl.BlockSpec((_R, D), lambda i: (i, 0)),
            pl.BlockSpec((D, D), lambda i: (0, 0)),
            pl.BlockSpec((1, D), lambda i: (0, 0)),
            pl.BlockSpec((D, D), lambda i: (0, 0)),
        ],
        out_specs=pl.BlockSpec((_R, D), lambda i: (i, 0)),
        out_shape=jax.ShapeDtypeStruct((NP, D), jnp.float32),
    )(x_pad, w_pre, b_pre.reshape(1, D), w1)


def _mid_body(c_ref, acc_ref, deg_ref, g_ref, b_ref, w_ref, bias_ref, out_ref):
    a = acc_ref[0] + acc_ref[1]
    deg = jnp.maximum(deg_ref[0] + deg_ref[1], 1.0)
    y = (a / deg[:, None]) * c_ref[0]
    m = jnp.mean(y, axis=-1, keepdims=True)
    d = y - m
    v = jnp.mean(d * d, axis=-1, keepdims=True)
    t = d * lax.rsqrt(v + 1e-5) * g_ref[...] + b_ref[...]
    t = jnp.maximum(t, 0.0)
    out_ref[...] = (
        jnp.dot(t, w_ref[...], preferred_element_type=jnp.float32)
        + bias_ref[...]
    )


def _tc_mid(cscal, acc, deg2, gvec, bvec, w, bias):
    return pl.pallas_call(
        _mid_body,
        grid=(NP // _R,),
        in_specs=[
            pl.BlockSpec(memory_space=pltpu.SMEM),
            pl.BlockSpec((NC, _R, D), lambda i: (0, i, 0)),
            pl.BlockSpec((NC, _R), lambda i: (0, i)),
            pl.BlockSpec((1, D), lambda i: (0, 0)),
            pl.BlockSpec((1, D), lambda i: (0, 0)),
            pl.BlockSpec((D, D), lambda i: (0, 0)),
            pl.BlockSpec((1, D), lambda i: (0, 0)),
        ],
        out_specs=pl.BlockSpec((_R, D), lambda i: (i, 0)),
        out_shape=jax.ShapeDtypeStruct((NP, D), jnp.float32),
    )(cscal, acc, deg2, gvec, bvec, w, bias)


def kernel(x, edge_index, gumbel_softmax_sample_ret_list,
           sample_candidate_index_list, W_pre, b_pre, W1, W2, g1, beta1,
           g2, beta2, W_post, b_post):
    g = gumbel_softmax_sample_ret_list
    sidx = sample_candidate_index_list
    c1 = g[0, 0, sidx[0]]
    c2 = g[1, 0, sidx[1]]
    c3 = g[2, 0, sidx[2]]
    c4 = g[0, 1, sidx[3]]
    c5 = g[1, 1, sidx[4]]
    c6 = g[2, 1, sidx[5]]

    pad = E_PAD - E
    src2 = jnp.concatenate(
        [edge_index[0], jnp.zeros((pad,), jnp.int32)]).reshape(NROWI, CHUNK)
    pad_dst = N + jnp.arange(pad, dtype=jnp.int32) % (NP - N)
    dst2 = jnp.concatenate(
        [edge_index[1], pad_dst]).reshape(NROWI, CHUNK)
    idx2 = jnp.stack([src2, dst2], axis=1)
    x_pad = jnp.pad(x, ((0, NP - N), (0, 0)))
    zrows = jnp.zeros((ROWS_PT, D), jnp.float32)
    zrow = jnp.zeros((ROWS_PT,), jnp.float32)

    p1 = _tc_pre(x_pad, W_pre, b_pre, W1)
    acc1, deg2 = _sc_conv_deg(p1, idx2, zrows, zrow)
    p2 = _tc_mid(c1.reshape(1), acc1, deg2, (g1 * c2).reshape(1, D),
                 (beta1 * c2).reshape(1, D), W2 * c3,
                 jnp.zeros((1, D), jnp.float32))
    acc2 = _sc_conv(p2, idx2, zrows)
    out = _tc_mid(c4.reshape(1), acc2, deg2, (g2 * c5).reshape(1, D),
                  (beta2 * c5).reshape(1, D), W_post * c6,
                  b_post.reshape(1, D))
    return out[:N]


# speedup vs baseline: 1.1089x; 1.1089x over previous
import functools

import jax
import jax.numpy as jnp
from jax import lax
from jax.experimental import pallas as pl
from jax.experimental.pallas import tpu as pltpu
from jax.experimental.pallas import tpu_sc as plsc

N = 10000
E = 320000
D = 128

NC = 2
NS = 16
NW = NC * NS
CHUNK = 112
NP = 10240
ROWS_PT = NP // NS
NBUF = 3
K = NBUF - 1
NIB = 2 * NBUF
NCH0 = 126
NCH1 = 54
E_PAD = NS * (NCH0 + NCH1) * CHUNK
NROWI = E_PAD // CHUNK

_MESH = dict(core_axis_name="c", subcore_axis_name="s", num_cores=NC,
             num_subcores=NS)


def _conv_impl(with_deg, p_hbm, idx2_hbm, zrows_hbm, zrow_hbm,
               acc_hbm, deg_hbm,
               acc_sp, deg_sp, idx_ring, rows, ones_v, sem_g, sem_s, sem_i):
    c = lax.axis_index("c")
    s = lax.axis_index("s")
    ebase = jnp.where(c == 0, s * NCH0, NS * NCH0 + s * NCH1)
    nchunk = jnp.where(c == 0, NCH0, NCH1)

    pltpu.sync_copy(zrows_hbm, acc_sp.at[pl.ds(s * ROWS_PT, ROWS_PT)])
    if with_deg:
        pltpu.sync_copy(zrow_hbm, deg_sp.at[pl.ds(s * ROWS_PT, ROWS_PT)])
        for j in range(CHUNK // 16):
            ones_v[pl.ds(j * 16, 16)] = jnp.full((16,), 1.0, jnp.float32)
    plsc.subcore_barrier()

    def prefetch_idx(jj, slot):
        pltpu.async_copy(idx2_hbm.at[ebase + jj], idx_ring.at[slot],
                         sem_i[slot])

    def issue_gather(jj_unused, b, slot):
        pltpu.make_async_copy(idx2_hbm.at[ebase], idx_ring.at[slot],
                              sem_i[slot]).wait()
        pltpu.async_copy(p_hbm.at[idx_ring.at[slot, 0]], rows[b], sem_g[b])

    def wait_scatter(b, slot):
        pltpu.make_async_copy(rows[b], acc_sp.at[idx_ring.at[slot, 1]],
                              sem_s[b]).wait()
        if with_deg:
            pltpu.make_async_copy(ones_v, deg_sp.at[idx_ring.at[slot, 1]],
                                  sem_s[b]).wait()

    def step(jj, u, drain, pref, gath):
        b = u % NBUF
        if drain:
            wait_scatter((u - 1) % NBUF, (u - 1) % NIB)
        if pref:
            prefetch_idx(jj + NBUF, (u + NBUF) % NIB)
        pltpu.make_async_copy(p_hbm.at[idx_ring.at[u, 0]], rows[b],
                              sem_g[b]).wait()
        pltpu.async_copy(rows[b], acc_sp.at[idx_ring.at[u, 1]], sem_s[b],
                         add=True)
        if with_deg:
            pltpu.async_copy(ones_v, deg_sp.at[idx_ring.at[u, 1]], sem_s[b],
                             add=True)
        if gath:
            issue_gather(jj + K, (b + K) % NBUF, (u + K) % NIB)

    for m in range(NBUF):
        prefetch_idx(m, m)
    for b in range(K):
        issue_gather(b, b, b)

    for u in range(NIB):
        step(u, u, drain=(u > 0), pref=True, gath=True)

    def round_body(r, _):
        jj0 = r * NIB
        for u in range(NIB):
            step(jj0 + u, u, drain=True, pref=True, gath=True)
        return 0

    lax.fori_loop(1, nchunk // NIB - 1, round_body, 0)

    jl = nchunk - NIB
    for u in range(NIB):
        step(jl + u, u, drain=True, pref=(u < NIB - NBUF),
             gath=(u < NIB - K))
    wait_scatter((NIB - 1) % NBUF, NIB - 1)
    plsc.subcore_barrier()

    r = pl.ds(s * ROWS_PT, ROWS_PT)
    pltpu.sync_copy(acc_sp.at[r], acc_hbm.at[c, r])
    if with_deg:
        pltpu.sync_copy(deg_sp.at[r], deg_hbm.at[c, r])


def _common_scratch():
    return [
        pltpu.VMEM((NIB, 2, CHUNK), jnp.int32),
        [pltpu.VMEM((CHUNK, D), jnp.float32) for _ in range(NBUF)],
        [pltpu.SemaphoreType.DMA for _ in range(NBUF)],
        [pltpu.SemaphoreType.DMA for _ in range(NBUF)],
        [pltpu.SemaphoreType.DMA for _ in range(NIB)],
    ]


@functools.partial(
    pl.kernel,
    out_type=(jax.ShapeDtypeStruct((NC, NP, D), jnp.float32),
              jax.ShapeDtypeStruct((NC, NP), jnp.float32)),
    mesh=plsc.VectorSubcoreMesh(**_MESH),
    scratch_types=[pltpu.VMEM_SHARED((NP, D), jnp.float32),
                   pltpu.VMEM_SHARED((NP,), jnp.float32),
                   pltpu.VMEM((CHUNK,), jnp.float32)] + _common_scratch(),
)
def _sc_conv_deg(p_hbm, idx2_hbm, zrows_hbm, zrow_hbm, acc_hbm, deg_hbm,
                 acc_sp, deg_sp, ones_v, idx_ring, rows, sem_g, sem_s, sem_i):
    _conv_impl(True, p_hbm, idx2_hbm, zrows_hbm, zrow_hbm, acc_hbm, deg_hbm,
               acc_sp, deg_sp, idx_ring, rows, ones_v, sem_g, sem_s, sem_i)


@functools.partial(
    pl.kernel,
    out_type=jax.ShapeDtypeStruct((NC, NP, D), jnp.float32),
    mesh=plsc.VectorSubcoreMesh(**_MESH),
    scratch_types=[pltpu.VMEM_SHARED((NP, D), jnp.float32)]
    + _common_scratch(),
)
def _sc_conv(p_hbm, idx2_hbm, zrows_hbm, acc_hbm,
             acc_sp, idx_ring, rows, sem_g, sem_s, sem_i):
    _conv_impl(False, p_hbm, idx2_hbm, zrows_hbm, None, acc_hbm, None,
               acc_sp, None, idx_ring, rows, None, sem_g, sem_s, sem_i)


_R = 1280


def _pre_body(x_ref, wpre_ref, bpre_ref, w1_ref, out_ref):
    h = jnp.dot(x_ref[...], wpre_ref[...], preferred_element_type=jnp.float32)
    h = h + bpre_ref[...]
    out_ref[...] = jnp.dot(h, w1_ref[...], preferred_element_type=jnp.float32)


def _tc_pre(x_pad, w_pre, b_pre, w1):
    return pl.pallas_call(
        _pre_body,
        grid=(NP // _R,),
        in_specs=[
            pl.BlockSpec((_R, D), lambda i: (i, 0)),
            pl.BlockSpec((D, D), lambda i: (0, 0)),
            pl.BlockSpec((1, D), lambda i: (0, 0)),
            pl.BlockSpec((D, D), lambda i: (0, 0)),
        ],
        out_specs=pl.BlockSpec((_R, D), lambda i: (i, 0)),
        out_shape=jax.ShapeDtypeStruct((NP, D), jnp.float32),
    )(x_pad, w_pre, b_pre.reshape(1, D), w1)


def _mid_body(c_ref, acc_ref, deg_ref, g_ref, b_ref, w_ref, bias_ref, out_ref):
    a = acc_ref[0] + acc_ref[1]
    deg = jnp.maximum(deg_ref[0] + deg_ref[1], 1.0)
    y = (a / deg[:, None]) * c_ref[0]
    m = jnp.mean(y, axis=-1, keepdims=True)
    d = y - m
    v = jnp.mean(d * d, axis=-1, keepdims=True)
    t = d * lax.rsqrt(v + 1e-5) * g_ref[...] + b_ref[...]
    t = jnp.maximum(t, 0.0)
    out_ref[...] = (
        jnp.dot(t, w_ref[...], preferred_element_type=jnp.float32)
        + bias_ref[...]
    )


def _tc_mid(cscal, acc, deg2, gvec, bvec, w, bias):
    return pl.pallas_call(
        _mid_body,
        grid=(NP // _R,),
        in_specs=[
            pl.BlockSpec(memory_space=pltpu.SMEM),
            pl.BlockSpec((NC, _R, D), lambda i: (0, i, 0)),
            pl.BlockSpec((NC, _R), lambda i: (0, i)),
            pl.BlockSpec((1, D), lambda i: (0, 0)),
            pl.BlockSpec((1, D), lambda i: (0, 0)),
            pl.BlockSpec((D, D), lambda i: (0, 0)),
            pl.BlockSpec((1, D), lambda i: (0, 0)),
        ],
        out_specs=pl.BlockSpec((_R, D), lambda i: (i, 0)),
        out_shape=jax.ShapeDtypeStruct((NP, D), jnp.float32),
    )(cscal, acc, deg2, gvec, bvec, w, bias)


def kernel(x, edge_index, gumbel_softmax_sample_ret_list,
           sample_candidate_index_list, W_pre, b_pre, W1, W2, g1, beta1,
           g2, beta2, W_post, b_post):
    g = gumbel_softmax_sample_ret_list
    sidx = sample_candidate_index_list
    c1 = g[0, 0, sidx[0]]
    c2 = g[1, 0, sidx[1]]
    c3 = g[2, 0, sidx[2]]
    c4 = g[0, 1, sidx[3]]
    c5 = g[1, 1, sidx[4]]
    c6 = g[2, 1, sidx[5]]

    pad = E_PAD - E
    src2 = jnp.concatenate(
        [edge_index[0], jnp.zeros((pad,), jnp.int32)]).reshape(NROWI, CHUNK)
    pad_dst = N + jnp.arange(pad, dtype=jnp.int32) % (NP - N)
    dst2 = jnp.concatenate(
        [edge_index[1], pad_dst]).reshape(NROWI, CHUNK)
    idx2 = jnp.stack([src2, dst2], axis=1)
    x_pad = jnp.pad(x, ((0, NP - N), (0, 0)))
    zrows = jnp.zeros((ROWS_PT, D), jnp.float32)
    zrow = jnp.zeros((ROWS_PT,), jnp.float32)

    p1 = _tc_pre(x_pad, W_pre, b_pre, W1)
    acc1, deg2 = _sc_conv_deg(p1, idx2, zrows, zrow)
    p2 = _tc_mid(c1.reshape(1), acc1, deg2, (g1 * c2).reshape(1, D),
                 (beta1 * c2).reshape(1, D), W2 * c3,
                 jnp.zeros((1, D), jnp.float32))
    acc2 = _sc_conv(p2, idx2, zrows)
    out = _tc_mid(c4.reshape(1), acc2, deg2, (g2 * c5).reshape(1, D),
                  (beta2 * c5).reshape(1, D), W_post * c6,
                  b_post.reshape(1, D))
    return out[:N]


# speedup vs baseline: 1.2213x; 1.1014x over previous
import functools

import jax
import jax.numpy as jnp
from jax import lax
from jax.experimental import pallas as pl
from jax.experimental.pallas import tpu as pltpu
from jax.experimental.pallas import tpu_sc as plsc

N = 10000
E = 320000
D = 128

NC = 2
NS = 16
NW = NC * NS
CHUNK = 112
NP = 10240
ROWS_PT = NP // NS
NBUF = 3
K = NBUF - 1
NIB = 2 * NBUF
NCH0 = 162
NCH1 = 18
E_PAD = NS * (NCH0 + NCH1) * CHUNK
NROWI = E_PAD // CHUNK

_MESH = dict(core_axis_name="c", subcore_axis_name="s", num_cores=NC,
             num_subcores=NS)


def _conv_impl(with_deg, p_hbm, idx2_hbm, zrows_hbm, zrow_hbm,
               acc_hbm, deg_hbm,
               acc_sp, deg_sp, idx_ring, rows, ones_v, sem_g, sem_s, sem_i):
    c = lax.axis_index("c")
    s = lax.axis_index("s")
    ebase = jnp.where(c == 0, s * NCH0, NS * NCH0 + s * NCH1)
    nchunk = jnp.where(c == 0, NCH0, NCH1)

    pltpu.sync_copy(zrows_hbm, acc_sp.at[pl.ds(s * ROWS_PT, ROWS_PT)])
    if with_deg:
        pltpu.sync_copy(zrow_hbm, deg_sp.at[pl.ds(s * ROWS_PT, ROWS_PT)])
        for j in range(CHUNK // 16):
            ones_v[pl.ds(j * 16, 16)] = jnp.full((16,), 1.0, jnp.float32)
    plsc.subcore_barrier()

    def prefetch_idx(jj, slot):
        pltpu.async_copy(idx2_hbm.at[ebase + jj], idx_ring.at[slot],
                         sem_i[slot])

    def issue_gather(jj_unused, b, slot):
        pltpu.make_async_copy(idx2_hbm.at[ebase], idx_ring.at[slot],
                              sem_i[slot]).wait()
        pltpu.async_copy(p_hbm.at[idx_ring.at[slot, 0]], rows[b], sem_g[b])

    def wait_scatter(b, slot):
        pltpu.make_async_copy(rows[b], acc_sp.at[idx_ring.at[slot, 1]],
                              sem_s[b]).wait()
        if with_deg:
            pltpu.make_async_copy(ones_v, deg_sp.at[idx_ring.at[slot, 1]],
                                  sem_s[b]).wait()

    def step(jj, u, drain, pref, gath):
        b = u % NBUF
        if drain:
            wait_scatter((u - 1) % NBUF, (u - 1) % NIB)
        if pref:
            prefetch_idx(jj + NBUF, (u + NBUF) % NIB)
        pltpu.make_async_copy(p_hbm.at[idx_ring.at[u, 0]], rows[b],
                              sem_g[b]).wait()
        pltpu.async_copy(rows[b], acc_sp.at[idx_ring.at[u, 1]], sem_s[b],
                         add=True)
        if with_deg:
            pltpu.async_copy(ones_v, deg_sp.at[idx_ring.at[u, 1]], sem_s[b],
                             add=True)
        if gath:
            issue_gather(jj + K, (b + K) % NBUF, (u + K) % NIB)

    for m in range(NBUF):
        prefetch_idx(m, m)
    for b in range(K):
        issue_gather(b, b, b)

    for u in range(NIB):
        step(u, u, drain=(u > 0), pref=True, gath=True)

    def round_body(r, _):
        jj0 = r * NIB
        for u in range(NIB):
            step(jj0 + u, u, drain=True, pref=True, gath=True)
        return 0

    lax.fori_loop(1, nchunk // NIB - 1, round_body, 0)

    jl = nchunk - NIB
    for u in range(NIB):
        step(jl + u, u, drain=True, pref=(u < NIB - NBUF),
             gath=(u < NIB - K))
    wait_scatter((NIB - 1) % NBUF, NIB - 1)
    plsc.subcore_barrier()

    r = pl.ds(s * ROWS_PT, ROWS_PT)
    pltpu.sync_copy(acc_sp.at[r], acc_hbm.at[c, r])
    if with_deg:
        pltpu.sync_copy(deg_sp.at[r], deg_hbm.at[c, r])


def _common_scratch():
    return [
        pltpu.VMEM((NIB, 2, CHUNK), jnp.int32),
        [pltpu.VMEM((CHUNK, D), jnp.float32) for _ in range(NBUF)],
        [pltpu.SemaphoreType.DMA for _ in range(NBUF)],
        [pltpu.SemaphoreType.DMA for _ in range(NBUF)],
        [pltpu.SemaphoreType.DMA for _ in range(NIB)],
    ]


@functools.partial(
    pl.kernel,
    out_type=(jax.ShapeDtypeStruct((NC, NP, D), jnp.float32),
              jax.ShapeDtypeStruct((NC, NP), jnp.float32)),
    mesh=plsc.VectorSubcoreMesh(**_MESH),
    scratch_types=[pltpu.VMEM_SHARED((NP, D), jnp.float32),
                   pltpu.VMEM_SHARED((NP,), jnp.float32),
                   pltpu.VMEM((CHUNK,), jnp.float32)] + _common_scratch(),
)
def _sc_conv_deg(p_hbm, idx2_hbm, zrows_hbm, zrow_hbm, acc_hbm, deg_hbm,
                 acc_sp, deg_sp, ones_v, idx_ring, rows, sem_g, sem_s, sem_i):
    _conv_impl(True, p_hbm, idx2_hbm, zrows_hbm, zrow_hbm, acc_hbm, deg_hbm,
               acc_sp, deg_sp, idx_ring, rows, ones_v, sem_g, sem_s, sem_i)


@functools.partial(
    pl.kernel,
    out_type=jax.ShapeDtypeStruct((NC, NP, D), jnp.float32),
    mesh=plsc.VectorSubcoreMesh(**_MESH),
    scratch_types=[pltpu.VMEM_SHARED((NP, D), jnp.float32)]
    + _common_scratch(),
)
def _sc_conv(p_hbm, idx2_hbm, zrows_hbm, acc_hbm,
             acc_sp, idx_ring, rows, sem_g, sem_s, sem_i):
    _conv_impl(False, p_hbm, idx2_hbm, zrows_hbm, None, acc_hbm, None,
               acc_sp, None, idx_ring, rows, None, sem_g, sem_s, sem_i)


_R = 1280


def _pre_body(x_ref, wpre_ref, bpre_ref, w1_ref, out_ref):
    h = jnp.dot(x_ref[...], wpre_ref[...], preferred_element_type=jnp.float32)
    h = h + bpre_ref[...]
    out_ref[...] = jnp.dot(h, w1_ref[...], preferred_element_type=jnp.float32)


def _tc_pre(x_pad, w_pre, b_pre, w1):
    return pl.pallas_call(
        _pre_body,
        grid=(NP // _R,),
        in_specs=[
            pl.BlockSpec((_R, D), lambda i: (i, 0)),
            pl.BlockSpec((D, D), lambda i: (0, 0)),
            pl.BlockSpec((1, D), lambda i: (0, 0)),
            pl.BlockSpec((D, D), lambda i: (0, 0)),
        ],
        out_specs=pl.BlockSpec((_R, D), lambda i: (i, 0)),
        out_shape=jax.ShapeDtypeStruct((NP, D), jnp.float32),
    )(x_pad, w_pre, b_pre.reshape(1, D), w1)


def _mid_body(c_ref, acc_ref, deg_ref, g_ref, b_ref, w_ref, bias_ref, out_ref):
    a = acc_ref[0] + acc_ref[1]
    deg = jnp.maximum(deg_ref[0] + deg_ref[1], 1.0)
    y = (a / deg[:, None]) * c_ref[0]
    m = jnp.mean(y, axis=-1, keepdims=True)
    d = y - m
    v = jnp.mean(d * d, axis=-1, keepdims=True)
    t = d * lax.rsqrt(v + 1e-5) * g_ref[...] + b_ref[...]
    t = jnp.maximum(t, 0.0)
    out_ref[...] = (
        jnp.dot(t, w_ref[...], preferred_element_type=jnp.float32)
        + bias_ref[...]
    )


def _tc_mid(cscal, acc, deg2, gvec, bvec, w, bias):
    return pl.pallas_call(
        _mid_body,
        grid=(NP // _R,),
        in_specs=[
            pl.BlockSpec(memory_space=pltpu.SMEM),
            pl.BlockSpec((NC, _R, D), lambda i: (0, i, 0)),
            pl.BlockSpec((NC, _R), lambda i: (0, i)),
            pl.BlockSpec((1, D), lambda i: (0, 0)),
            pl.BlockSpec((1, D), lambda i: (0, 0)),
            pl.BlockSpec((D, D), lambda i: (0, 0)),
            pl.BlockSpec((1, D), lambda i: (0, 0)),
        ],
        out_specs=pl.BlockSpec((_R, D), lambda i: (i, 0)),
        out_shape=jax.ShapeDtypeStruct((NP, D), jnp.float32),
    )(cscal, acc, deg2, gvec, bvec, w, bias)


def kernel(x, edge_index, gumbel_softmax_sample_ret_list,
           sample_candidate_index_list, W_pre, b_pre, W1, W2, g1, beta1,
           g2, beta2, W_post, b_post):
    g = gumbel_softmax_sample_ret_list
    sidx = sample_candidate_index_list
    c1 = g[0, 0, sidx[0]]
    c2 = g[1, 0, sidx[1]]
    c3 = g[2, 0, sidx[2]]
    c4 = g[0, 1, sidx[3]]
    c5 = g[1, 1, sidx[4]]
    c6 = g[2, 1, sidx[5]]

    pad = E_PAD - E
    src2 = jnp.concatenate(
        [edge_index[0], jnp.zeros((pad,), jnp.int32)]).reshape(NROWI, CHUNK)
    pad_dst = N + jnp.arange(pad, dtype=jnp.int32) % (NP - N)
    dst2 = jnp.concatenate(
        [edge_index[1], pad_dst]).reshape(NROWI, CHUNK)
    idx2 = jnp.stack([src2, dst2], axis=1)
    x_pad = jnp.pad(x, ((0, NP - N), (0, 0)))
    zrows = jnp.zeros((ROWS_PT, D), jnp.float32)
    zrow = jnp.zeros((ROWS_PT,), jnp.float32)

    p1 = _tc_pre(x_pad, W_pre, b_pre, W1)
    acc1, deg2 = _sc_conv_deg(p1, idx2, zrows, zrow)
    p2 = _tc_mid(c1.reshape(1), acc1, deg2, (g1 * c2).reshape(1, D),
                 (beta1 * c2).reshape(1, D), W2 * c3,
                 jnp.zeros((1, D), jnp.float32))
    acc2 = _sc_conv(p2, idx2, zrows)
    out = _tc_mid(c4.reshape(1), acc2, deg2, (g2 * c5).reshape(1, D),
                  (beta2 * c5).reshape(1, D), W_post * c6,
                  b_post.reshape(1, D))
    return out[:N]


# speedup vs baseline: 1.2389x; 1.0145x over previous
import functools

import jax
import jax.numpy as jnp
from jax import lax
from jax.experimental import pallas as pl
from jax.experimental.pallas import tpu as pltpu
from jax.experimental.pallas import tpu_sc as plsc

N = 10000
E = 320000
D = 128

NC = 2
NS = 16
NW = NC * NS
CHUNK = 112
NP = 10240
ROWS_PT = NP // NS
NBUF = 3
K = NBUF - 1
NIB = 2 * NBUF
NCH0 = 162
NCH1 = 18
E_PAD = NS * (NCH0 + NCH1) * CHUNK
NROWI = E_PAD // CHUNK

_MESH = dict(core_axis_name="c", subcore_axis_name="s", num_cores=NC,
             num_subcores=NS)


def _conv_impl(with_deg, p_hbm, idx_hbm, zrows_hbm, zrow_hbm,
               acc_hbm, deg_hbm,
               acc_sp, deg_sp, idx_ring, rows, ones_v, sem_g, sem_s, sem_i):
    c = lax.axis_index("c")
    s = lax.axis_index("s")
    ebase = jnp.where(c == 0, s * NCH0, NS * NCH0 + s * NCH1)
    nchunk = jnp.where(c == 0, NCH0, NCH1)

    pltpu.sync_copy(zrows_hbm, acc_sp.at[pl.ds(s * ROWS_PT, ROWS_PT)])
    if with_deg:
        pltpu.sync_copy(zrow_hbm, deg_sp.at[pl.ds(s * ROWS_PT, ROWS_PT)])
        for j in range(CHUNK // 16):
            ones_v[pl.ds(j * 16, 16)] = jnp.full((16,), 1.0, jnp.float32)
    plsc.subcore_barrier()

    def prefetch_idx(jj, slot):
        base = (ebase + jj) * (2 * CHUNK)
        pltpu.async_copy(idx_hbm.at[pl.ds(base, CHUNK)],
                         idx_ring.at[slot, 0], sem_i[slot])
        pltpu.async_copy(idx_hbm.at[pl.ds(base + CHUNK, CHUNK)],
                         idx_ring.at[slot, 1], sem_i[slot])

    def wait_idx(slot):
        pltpu.make_async_copy(idx_hbm.at[pl.ds(0, CHUNK)],
                              idx_ring.at[slot, 0], sem_i[slot]).wait()
        pltpu.make_async_copy(idx_hbm.at[pl.ds(0, CHUNK)],
                              idx_ring.at[slot, 1], sem_i[slot]).wait()

    def issue_gather(b, slot):
        wait_idx(slot)
        pltpu.async_copy(p_hbm.at[idx_ring.at[slot, 0]], rows[b], sem_g[b])

    def wait_scatter(b, slot):
        pltpu.make_async_copy(rows[b], acc_sp.at[idx_ring.at[slot, 1]],
                              sem_s[b]).wait()
        if with_deg:
            pltpu.make_async_copy(ones_v, deg_sp.at[idx_ring.at[slot, 1]],
                                  sem_s[b]).wait()

    def step(jj, u, drain, pref, gath):
        b = u % NBUF
        if drain:
            wait_scatter((u - 1) % NBUF, (u - 1) % NIB)
        if pref:
            prefetch_idx(jj + NBUF, (u + NBUF) % NIB)
        pltpu.make_async_copy(p_hbm.at[idx_ring.at[u, 0]], rows[b],
                              sem_g[b]).wait()
        pltpu.async_copy(rows[b], acc_sp.at[idx_ring.at[u, 1]], sem_s[b],
                         add=True)
        if with_deg:
            pltpu.async_copy(ones_v, deg_sp.at[idx_ring.at[u, 1]], sem_s[b],
                             add=True)
        if gath:
            issue_gather((b + K) % NBUF, (u + K) % NIB)

    for m in range(NBUF):
        prefetch_idx(m, m)
    for b in range(K):
        issue_gather(b, b)

    for u in range(NIB):
        step(u, u, drain=(u > 0), pref=True, gath=True)

    def round_body(r, _):
        jj0 = r * NIB
        for u in range(NIB):
            step(jj0 + u, u, drain=True, pref=True, gath=True)
        return 0

    lax.fori_loop(1, nchunk // NIB - 1, round_body, 0)

    jl = nchunk - NIB
    for u in range(NIB):
        step(jl + u, u, drain=True, pref=(u < NIB - NBUF),
             gath=(u < NIB - K))
    wait_scatter((NIB - 1) % NBUF, NIB - 1)
    plsc.subcore_barrier()

    r = pl.ds(s * ROWS_PT, ROWS_PT)
    pltpu.sync_copy(acc_sp.at[r], acc_hbm.at[c, r])
    if with_deg:
        pltpu.sync_copy(deg_sp.at[r], deg_hbm.at[c, r])


def _common_scratch():
    return [
        pltpu.VMEM((NIB, 2, CHUNK), jnp.int32),
        [pltpu.VMEM((CHUNK, D), jnp.float32) for _ in range(NBUF)],
        [pltpu.SemaphoreType.DMA for _ in range(NBUF)],
        [pltpu.SemaphoreType.DMA for _ in range(NBUF)],
        [pltpu.SemaphoreType.DMA for _ in range(NIB)],
    ]


@functools.partial(
    pl.kernel,
    out_type=(jax.ShapeDtypeStruct((NC, NP, D), jnp.float32),
              jax.ShapeDtypeStruct((NC, NP), jnp.float32)),
    mesh=plsc.VectorSubcoreMesh(**_MESH),
    scratch_types=[pltpu.VMEM_SHARED((NP, D), jnp.float32),
                   pltpu.VMEM_SHARED((NP,), jnp.float32),
                   pltpu.VMEM((CHUNK,), jnp.float32)] + _common_scratch(),
)
def _sc_conv_deg(p_hbm, idx_hbm, zrows_hbm, zrow_hbm, acc_hbm, deg_hbm,
                 acc_sp, deg_sp, ones_v, idx_ring, rows, sem_g, sem_s, sem_i):
    _conv_impl(True, p_hbm, idx_hbm, zrows_hbm, zrow_hbm, acc_hbm, deg_hbm,
               acc_sp, deg_sp, idx_ring, rows, ones_v, sem_g, sem_s, sem_i)


@functools.partial(
    pl.kernel,
    out_type=jax.ShapeDtypeStruct((NC, NP, D), jnp.float32),
    mesh=plsc.VectorSubcoreMesh(**_MESH),
    scratch_types=[pltpu.VMEM_SHARED((NP, D), jnp.float32)]
    + _common_scratch(),
)
def _sc_conv(p_hbm, idx_hbm, zrows_hbm, acc_hbm,
             acc_sp, idx_ring, rows, sem_g, sem_s, sem_i):
    _conv_impl(False, p_hbm, idx_hbm, zrows_hbm, None, acc_hbm, None,
               acc_sp, None, idx_ring, rows, None, sem_g, sem_s, sem_i)


_R = 2048


def _ln_relu(y, g_ref, b_ref):
    m = jnp.mean(y, axis=-1, keepdims=True)
    d = y - m
    v = jnp.mean(d * d, axis=-1, keepdims=True)
    t = d * lax.rsqrt(v + 1e-5) * g_ref[...] + b_ref[...]
    return jnp.maximum(t, 0.0)


def _mid1_body(c_ref, acc_ref, deg_ref, wpre_ref, bpre_ref, w1_ref,
               g_ref, b_ref, w2_ref, out_ref):
    wc = jnp.dot(wpre_ref[...], w1_ref[...],
                 preferred_element_type=jnp.float32)
    bc = jnp.dot(bpre_ref[...], w1_ref[...],
                 preferred_element_type=jnp.float32)
    a = acc_ref[0] + acc_ref[1]
    degt = deg_ref[0] + deg_ref[1]
    deg = jnp.maximum(degt, 1.0)
    adeg = a / deg[:, None]
    y = (jnp.dot(adeg, wc, preferred_element_type=jnp.float32)
         + bc * jnp.minimum(degt, 1.0)[:, None]) * c_ref[0]
    t = _ln_relu(y, g_ref, b_ref)
    out_ref[...] = jnp.dot(t, w2_ref[...], preferred_element_type=jnp.float32)


def _tc_mid1(cscal, acc, deg2, w_pre, b_pre, w1, gvec, bvec, w2c3):
    return pl.pallas_call(
        _mid1_body,
        grid=(NP // _R,),
        in_specs=[
            pl.BlockSpec(memory_space=pltpu.SMEM),
            pl.BlockSpec((NC, _R, D), lambda i: (0, i, 0)),
            pl.BlockSpec((NC, _R), lambda i: (0, i)),
            pl.BlockSpec((D, D), lambda i: (0, 0)),
            pl.BlockSpec((1, D), lambda i: (0, 0)),
            pl.BlockSpec((D, D), lambda i: (0, 0)),
            pl.BlockSpec((1, D), lambda i: (0, 0)),
            pl.BlockSpec((1, D), lambda i: (0, 0)),
            pl.BlockSpec((D, D), lambda i: (0, 0)),
        ],
        out_specs=pl.BlockSpec((_R, D), lambda i: (i, 0)),
        out_shape=jax.ShapeDtypeStruct((N, D), jnp.float32),
    )(cscal, acc, deg2, w_pre, b_pre, w1, gvec, bvec, w2c3)


def _mid2_body(c_ref, acc_ref, deg_ref, g_ref, b_ref, w_ref, bias_ref,
               out_ref):
    a = acc_ref[0] + acc_ref[1]
    deg = jnp.maximum(deg_ref[0] + deg_ref[1], 1.0)
    y = (a / deg[:, None]) * c_ref[0]
    t = _ln_relu(y, g_ref, b_ref)
    out_ref[...] = (
        jnp.dot(t, w_ref[...], preferred_element_type=jnp.float32)
        + bias_ref[...]
    )


def _tc_mid2(cscal, acc, deg2, gvec, bvec, w, bias):
    return pl.pallas_call(
        _mid2_body,
        grid=(NP // _R,),
        in_specs=[
            pl.BlockSpec(memory_space=pltpu.SMEM),
            pl.BlockSpec((NC, _R, D), lambda i: (0, i, 0)),
            pl.BlockSpec((NC, _R), lambda i: (0, i)),
            pl.BlockSpec((1, D), lambda i: (0, 0)),
            pl.BlockSpec((1, D), lambda i: (0, 0)),
            pl.BlockSpec((D, D), lambda i: (0, 0)),
            pl.BlockSpec((1, D), lambda i: (0, 0)),
        ],
        out_specs=pl.BlockSpec((_R, D), lambda i: (i, 0)),
        out_shape=jax.ShapeDtypeStruct((N, D), jnp.float32),
    )(cscal, acc, deg2, gvec, bvec, w, bias)


def kernel(x, edge_index, gumbel_softmax_sample_ret_list,
           sample_candidate_index_list, W_pre, b_pre, W1, W2, g1, beta1,
           g2, beta2, W_post, b_post):
    g = gumbel_softmax_sample_ret_list
    sidx = sample_candidate_index_list
    c1 = g[0, 0, sidx[0]]
    c2 = g[1, 0, sidx[1]]
    c3 = g[2, 0, sidx[2]]
    c4 = g[0, 1, sidx[3]]
    c5 = g[1, 1, sidx[4]]
    c6 = g[2, 1, sidx[5]]

    pad = E_PAD - E
    pad_dst = N + jnp.arange(pad, dtype=jnp.int32) % (NP - N)
    pad_blk = jnp.stack([jnp.zeros((pad,), jnp.int32), pad_dst])
    idx1d = (jnp.concatenate([edge_index, pad_blk], axis=1)
             .reshape(2, NROWI, CHUNK).transpose(1, 0, 2).reshape(-1))
    zrows = jnp.zeros((ROWS_PT, D), jnp.float32)
    zrow = jnp.zeros((ROWS_PT,), jnp.float32)

    acc1, deg2 = _sc_conv_deg(x, idx1d, zrows, zrow)
    p2 = _tc_mid1(c1.reshape(1), acc1, deg2, W_pre, b_pre.reshape(1, D), W1,
                  (g1 * c2).reshape(1, D), (beta1 * c2).reshape(1, D),
                  W2 * c3)
    acc2 = _sc_conv(p2, idx1d, zrows)
    out = _tc_mid2(c4.reshape(1), acc2, deg2, (g2 * c5).reshape(1, D),
                   (beta2 * c5).reshape(1, D), W_post * c6,
                   b_post.reshape(1, D))
    return out


# speedup vs baseline: 1.2833x; 1.0358x over previous
import functools

import jax
import jax.numpy as jnp
from jax import lax
from jax.experimental import pallas as pl
from jax.experimental.pallas import tpu as pltpu
from jax.experimental.pallas import tpu_sc as plsc

N = 10000
E = 320000
D = 128

NC = 2
NS = 16
NW = NC * NS
CHUNK = 112
NP = 10240
ROWS_PT = NP // NS
NBUF = 3
K = NBUF - 1
NIB = 2 * NBUF
NCH0 = 162
NCH1 = 18
E_PAD = NS * (NCH0 + NCH1) * CHUNK
NROWI = E_PAD // CHUNK

_MESH = dict(core_axis_name="c", subcore_axis_name="s", num_cores=NC,
             num_subcores=NS)


def _conv_impl(with_deg, p_hbm, idx_hbm, zrows_hbm, zrow_hbm,
               acc_hbm, deg_hbm,
               acc_sp, deg_sp, idx_ring, rows, ones_v, sem_g, sem_s, sem_i):
    c = lax.axis_index("c")
    s = lax.axis_index("s")
    ebase = jnp.where(c == 0, s * NCH0, NS * NCH0 + s * NCH1)
    nchunk = jnp.where(c == 0, NCH0, NCH1)

    pltpu.sync_copy(zrows_hbm, acc_sp.at[pl.ds(s * ROWS_PT, ROWS_PT)])
    if with_deg:
        pltpu.sync_copy(zrow_hbm, deg_sp.at[pl.ds(s * ROWS_PT, ROWS_PT)])
        for j in range(CHUNK // 16):
            ones_v[pl.ds(j * 16, 16)] = jnp.full((16,), 1.0, jnp.float32)
    plsc.subcore_barrier()

    def prefetch_idx(jj, slot):
        base = (ebase + jj) * CHUNK
        pltpu.async_copy(idx_hbm.at[pl.ds(base, CHUNK)],
                         idx_ring.at[slot, 0], sem_i[slot])
        pltpu.async_copy(idx_hbm.at[pl.ds(E_PAD + base, CHUNK)],
                         idx_ring.at[slot, 1], sem_i[slot])

    def wait_idx(slot):
        pltpu.make_async_copy(idx_hbm.at[pl.ds(0, CHUNK)],
                              idx_ring.at[slot, 0], sem_i[slot]).wait()
        pltpu.make_async_copy(idx_hbm.at[pl.ds(0, CHUNK)],
                              idx_ring.at[slot, 1], sem_i[slot]).wait()

    def issue_gather(b, slot):
        wait_idx(slot)
        pltpu.async_copy(p_hbm.at[idx_ring.at[slot, 0]], rows[b], sem_g[b])

    def wait_scatter(b, slot):
        pltpu.make_async_copy(rows[b], acc_sp.at[idx_ring.at[slot, 1]],
                              sem_s[b]).wait()
        if with_deg:
            pltpu.make_async_copy(ones_v, deg_sp.at[idx_ring.at[slot, 1]],
                                  sem_s[b]).wait()

    def step(jj, u, drain, pref, gath):
        b = u % NBUF
        if drain:
            wait_scatter((u - 1) % NBUF, (u - 1) % NIB)
        if pref:
            prefetch_idx(jj + NBUF, (u + NBUF) % NIB)
        pltpu.make_async_copy(p_hbm.at[idx_ring.at[u, 0]], rows[b],
                              sem_g[b]).wait()
        pltpu.async_copy(rows[b], acc_sp.at[idx_ring.at[u, 1]], sem_s[b],
                         add=True)
        if with_deg:
            pltpu.async_copy(ones_v, deg_sp.at[idx_ring.at[u, 1]], sem_s[b],
                             add=True)
        if gath:
            issue_gather((b + K) % NBUF, (u + K) % NIB)

    for m in range(NBUF):
        prefetch_idx(m, m)
    for b in range(K):
        issue_gather(b, b)

    for u in range(NIB):
        step(u, u, drain=(u > 0), pref=True, gath=True)

    def round_body(r, _):
        jj0 = r * NIB
        for u in range(NIB):
            step(jj0 + u, u, drain=True, pref=True, gath=True)
        return 0

    lax.fori_loop(1, nchunk // NIB - 1, round_body, 0)

    jl = nchunk - NIB
    for u in range(NIB):
        step(jl + u, u, drain=True, pref=(u < NIB - NBUF),
             gath=(u < NIB - K))
    wait_scatter((NIB - 1) % NBUF, NIB - 1)
    plsc.subcore_barrier()

    r = pl.ds(s * ROWS_PT, ROWS_PT)
    pltpu.sync_copy(acc_sp.at[r], acc_hbm.at[c, r])
    if with_deg:
        pltpu.sync_copy(deg_sp.at[r], deg_hbm.at[c, r])


def _common_scratch():
    return [
        pltpu.VMEM((NIB, 2, CHUNK), jnp.int32),
        [pltpu.VMEM((CHUNK, D), jnp.float32) for _ in range(NBUF)],
        [pltpu.SemaphoreType.DMA for _ in range(NBUF)],
        [pltpu.SemaphoreType.DMA for _ in range(NBUF)],
        [pltpu.SemaphoreType.DMA for _ in range(NIB)],
    ]


@functools.partial(
    pl.kernel,
    out_type=(jax.ShapeDtypeStruct((NC, NP, D), jnp.float32),
              jax.ShapeDtypeStruct((NC, NP), jnp.float32)),
    mesh=plsc.VectorSubcoreMesh(**_MESH),
    scratch_types=[pltpu.VMEM_SHARED((NP, D), jnp.float32),
                   pltpu.VMEM_SHARED((NP,), jnp.float32),
                   pltpu.VMEM((CHUNK,), jnp.float32)] + _common_scratch(),
)
def _sc_conv_deg(p_hbm, idx_hbm, zrows_hbm, zrow_hbm, acc_hbm, deg_hbm,
                 acc_sp, deg_sp, ones_v, idx_ring, rows, sem_g, sem_s, sem_i):
    _conv_impl(True, p_hbm, idx_hbm, zrows_hbm, zrow_hbm, acc_hbm, deg_hbm,
               acc_sp, deg_sp, idx_ring, rows, ones_v, sem_g, sem_s, sem_i)


@functools.partial(
    pl.kernel,
    out_type=jax.ShapeDtypeStruct((NC, NP, D), jnp.float32),
    mesh=plsc.VectorSubcoreMesh(**_MESH),
    scratch_types=[pltpu.VMEM_SHARED((NP, D), jnp.float32)]
    + _common_scratch(),
)
def _sc_conv(p_hbm, idx_hbm, zrows_hbm, acc_hbm,
             acc_sp, idx_ring, rows, sem_g, sem_s, sem_i):
    _conv_impl(False, p_hbm, idx_hbm, zrows_hbm, None, acc_hbm, None,
               acc_sp, None, idx_ring, rows, None, sem_g, sem_s, sem_i)


_R = 2048


def _ln_relu(y, g_ref, b_ref):
    m = jnp.mean(y, axis=-1, keepdims=True)
    d = y - m
    v = jnp.mean(d * d, axis=-1, keepdims=True)
    t = d * lax.rsqrt(v + 1e-5) * g_ref[...] + b_ref[...]
    return jnp.maximum(t, 0.0)


def _mid1_body(c_ref, acc_ref, deg_ref, wpre_ref, bpre_ref, w1_ref,
               g_ref, b_ref, w2_ref, out_ref):
    wc = jnp.dot(wpre_ref[...], w1_ref[...],
                 preferred_element_type=jnp.float32)
    bc = jnp.dot(bpre_ref[...], w1_ref[...],
                 preferred_element_type=jnp.float32)
    a = acc_ref[0] + acc_ref[1]
    degt = deg_ref[0] + deg_ref[1]
    deg = jnp.maximum(degt, 1.0)
    adeg = a / deg[:, None]
    y = (jnp.dot(adeg, wc, preferred_element_type=jnp.float32)
         + bc * jnp.minimum(degt, 1.0)[:, None]) * c_ref[0]
    t = _ln_relu(y, g_ref, b_ref)
    out_ref[...] = jnp.dot(t, w2_ref[...], preferred_element_type=jnp.float32)


def _tc_mid1(cscal, acc, deg2, w_pre, b_pre, w1, gvec, bvec, w2c3):
    return pl.pallas_call(
        _mid1_body,
        grid=(NP // _R,),
        in_specs=[
            pl.BlockSpec(memory_space=pltpu.SMEM),
            pl.BlockSpec((NC, _R, D), lambda i: (0, i, 0)),
            pl.BlockSpec((NC, _R), lambda i: (0, i)),
            pl.BlockSpec((D, D), lambda i: (0, 0)),
            pl.BlockSpec((1, D), lambda i: (0, 0)),
            pl.BlockSpec((D, D), lambda i: (0, 0)),
            pl.BlockSpec((1, D), lambda i: (0, 0)),
            pl.BlockSpec((1, D), lambda i: (0, 0)),
            pl.BlockSpec((D, D), lambda i: (0, 0)),
        ],
        out_specs=pl.BlockSpec((_R, D), lambda i: (i, 0)),
        out_shape=jax.ShapeDtypeStruct((N, D), jnp.float32),
    )(cscal, acc, deg2, w_pre, b_pre, w1, gvec, bvec, w2c3)


def _mid2_body(c_ref, acc_ref, deg_ref, g_ref, b_ref, w_ref, bias_ref,
               out_ref):
    a = acc_ref[0] + acc_ref[1]
    deg = jnp.maximum(deg_ref[0] + deg_ref[1], 1.0)
    y = (a / deg[:, None]) * c_ref[0]
    t = _ln_relu(y, g_ref, b_ref)
    out_ref[...] = (
        jnp.dot(t, w_ref[...], preferred_element_type=jnp.float32)
        + bias_ref[...]
    )


def _tc_mid2(cscal, acc, deg2, gvec, bvec, w, bias):
    return pl.pallas_call(
        _mid2_body,
        grid=(NP // _R,),
        in_specs=[
            pl.BlockSpec(memory_space=pltpu.SMEM),
            pl.BlockSpec((NC, _R, D), lambda i: (0, i, 0)),
            pl.BlockSpec((NC, _R), lambda i: (0, i)),
            pl.BlockSpec((1, D), lambda i: (0, 0)),
            pl.BlockSpec((1, D), lambda i: (0, 0)),
            pl.BlockSpec((D, D), lambda i: (0, 0)),
            pl.BlockSpec((1, D), lambda i: (0, 0)),
        ],
        out_specs=pl.BlockSpec((_R, D), lambda i: (i, 0)),
        out_shape=jax.ShapeDtypeStruct((N, D), jnp.float32),
    )(cscal, acc, deg2, gvec, bvec, w, bias)


def kernel(x, edge_index, gumbel_softmax_sample_ret_list,
           sample_candidate_index_list, W_pre, b_pre, W1, W2, g1, beta1,
           g2, beta2, W_post, b_post):
    g = gumbel_softmax_sample_ret_list
    sidx = sample_candidate_index_list
    c1 = g[0, 0, sidx[0]]
    c2 = g[1, 0, sidx[1]]
    c3 = g[2, 0, sidx[2]]
    c4 = g[0, 1, sidx[3]]
    c5 = g[1, 1, sidx[4]]
    c6 = g[2, 1, sidx[5]]

    pad = E_PAD - E
    pad_dst = N + jnp.arange(pad, dtype=jnp.int32) % (NP - N)
    idx1d = jnp.concatenate(
        [edge_index[0], jnp.zeros((pad,), jnp.int32), edge_index[1], pad_dst])
    zrows = jnp.zeros((ROWS_PT, D), jnp.float32)
    zrow = jnp.zeros((ROWS_PT,), jnp.float32)

    acc1, deg2 = _sc_conv_deg(x, idx1d, zrows, zrow)
    p2 = _tc_mid1(c1.reshape(1), acc1, deg2, W_pre, b_pre.reshape(1, D), W1,
                  (g1 * c2).reshape(1, D), (beta1 * c2).reshape(1, D),
                  W2 * c3)
    acc2 = _sc_conv(p2, idx1d, zrows)
    out = _tc_mid2(c4.reshape(1), acc2, deg2, (g2 * c5).reshape(1, D),
                   (beta2 * c5).reshape(1, D), W_post * c6,
                   b_post.reshape(1, D))
    return out
